# baseline (speedup 1.0000x reference)
import functools

import jax
import jax.numpy as jnp
from jax import lax
from jax.experimental import pallas as pl
from jax.experimental.pallas import tpu as pltpu
from jax.experimental.pallas import tpu_sc as plsc

MAX_LEN = 200
DIM = 64
ROW_WORDS = MAX_LEN * DIM
NUM_CORES = 2
NUM_SUBCORES = 16
NUM_WORKERS = NUM_CORES * NUM_SUBCORES
REP = 8
FIRE = 8
LANES = 16


SREP = 128


@functools.partial(jax.jit, static_argnums=(1,))
def _sc_broadcast(w_flat, batch):
    rows_per_w = batch // NUM_WORKERS
    chunk_words = SREP * ROW_WORDS
    chunks = rows_per_w // SREP
    rep_per_tile = SREP // NUM_SUBCORES
    total = batch * ROW_WORDS

    mesh = plsc.VectorSubcoreMesh(
        core_axis_name="c", subcore_axis_name="s",
        num_cores=NUM_CORES, num_subcores=NUM_SUBCORES)

    @functools.partial(
        pl.kernel,
        mesh=mesh,
        out_type=jax.ShapeDtypeStruct((total,), jnp.float32),
        scratch_types=[
            pltpu.VMEM((ROW_WORDS,), jnp.float32),
            pltpu.VMEM_SHARED((chunk_words,), jnp.float32),
            pltpu.SemaphoreType.DMA,
        ],
    )
    def k(w_hbm, out_hbm, w_v, shared, sem):
        cid = lax.axis_index("c")
        sid = lax.axis_index("s")
        wid = sid * NUM_CORES + cid
        pltpu.sync_copy(w_hbm, w_v)

        def relu_body(i, carry):
            w_v[pl.ds(i * LANES, LANES)] = jnp.maximum(
                w_v[pl.ds(i * LANES, LANES)], 0.0)
            return carry

        lax.fori_loop(0, ROW_WORDS // LANES, relu_body, 0)

        for r in range(rep_per_tile):
            pltpu.sync_copy(
                w_v,
                shared.at[pl.ds((sid * rep_per_tile + r) * ROW_WORDS,
                                ROW_WORDS)])
        plsc.subcore_barrier()

        base = wid * (rows_per_w * ROW_WORDS)
        copies = [
            pltpu.async_copy(
                shared,
                out_hbm.at[pl.ds(base + j * chunk_words, chunk_words)],
                sem)
            for j in range(chunks)
        ]
        for c in copies:
            c.wait()

    return k(w_flat)


def kernel(x, embed_weight):
    batch, seq = x.shape[0], x.shape[1]
    w_flat = embed_weight[:seq].reshape(-1)
    out = _sc_broadcast(w_flat, batch)
    return out.reshape(batch, seq, DIM)


# speedup vs baseline: 1.0012x; 1.0012x over previous
import functools

import jax
import jax.numpy as jnp
from jax import lax
from jax.experimental import pallas as pl
from jax.experimental.pallas import tpu as pltpu
from jax.experimental.pallas import tpu_sc as plsc

MAX_LEN = 200
DIM = 64
NUM_CORES = 2
NUM_SUBCORES = 16
NUM_WORKERS = NUM_CORES * NUM_SUBCORES
LANES = 16
SREP = 128


@functools.partial(jax.jit, static_argnums=(1,))
def _sc_broadcast(w, batch):
    rows_per_w = batch // NUM_WORKERS
    chunks = rows_per_w // SREP
    rep_per_tile = SREP // NUM_SUBCORES

    mesh = plsc.VectorSubcoreMesh(
        core_axis_name="c", subcore_axis_name="s",
        num_cores=NUM_CORES, num_subcores=NUM_SUBCORES)

    @functools.partial(
        pl.kernel,
        mesh=mesh,
        out_type=jax.ShapeDtypeStruct((batch, MAX_LEN, DIM), jnp.float32),
        scratch_types=[
            pltpu.VMEM((MAX_LEN, DIM), jnp.float32),
            pltpu.VMEM_SHARED((SREP, MAX_LEN, DIM), jnp.float32),
            pltpu.SemaphoreType.DMA,
        ],
        compiler_params=pltpu.CompilerParams(use_tc_tiling_on_sc=False),
    )
    def k(w_hbm, out_hbm, w_v, shared, sem):
        cid = lax.axis_index("c")
        sid = lax.axis_index("s")
        wid = sid * NUM_CORES + cid
        pltpu.sync_copy(w_hbm, w_v)

        def relu_body(r, carry):
            for cc in range(DIM // LANES):
                w_v[r, pl.ds(cc * LANES, LANES)] = jnp.maximum(
                    w_v[r, pl.ds(cc * LANES, LANES)], 0.0)
            return carry

        lax.fori_loop(0, MAX_LEN, relu_body, 0)

        for r in range(rep_per_tile):
            pltpu.sync_copy(w_v, shared.at[sid * rep_per_tile + r])
        plsc.subcore_barrier()

        base = wid * rows_per_w
        copies = [
            pltpu.async_copy(
                shared,
                out_hbm.at[pl.ds(base + j * SREP, SREP)],
                sem)
            for j in range(chunks)
        ]
        for c in copies:
            c.wait()

    return k(w)


def kernel(x, embed_weight):
    batch, seq = x.shape[0], x.shape[1]
    return _sc_broadcast(embed_weight[:seq], batch)


# speedup vs baseline: 8.3707x; 8.3608x over previous
import functools

import jax
import jax.numpy as jnp
from jax import lax
from jax.experimental import pallas as pl
from jax.experimental.pallas import tpu as pltpu
from jax.experimental.pallas import tpu_sc as plsc

MAX_LEN = 200
DIM = 64
BATCH = 16384
NUM_CORES = 2
NUM_SUBCORES = 16
NUM_WORKERS = NUM_CORES * NUM_SUBCORES
LANES = 16

SB = 5
DQ = 16
BW = 512
UNITS = (MAX_LEN // SB) * (DIM // DQ)
UNITS_PER_W = UNITS // NUM_WORKERS
NCHUNKS = BATCH // BW
FIRE = 8


@jax.jit
def _sc_pos_broadcast(w):
    mesh = plsc.VectorSubcoreMesh(
        core_axis_name="c", subcore_axis_name="s",
        num_cores=NUM_CORES, num_subcores=NUM_SUBCORES)

    @functools.partial(
        pl.kernel,
        mesh=mesh,
        out_type=jax.ShapeDtypeStruct((MAX_LEN, DIM, BATCH), jnp.float32),
        scratch_types=[
            pltpu.VMEM((MAX_LEN, DIM), jnp.float32),
            pltpu.VMEM((SB, DQ, BW), jnp.float32),
            pltpu.SemaphoreType.DMA,
        ],
        compiler_params=pltpu.CompilerParams(use_tc_tiling_on_sc=True),
    )
    def k(w_hbm, out_hbm, w_v, src, sem):
        wid = lax.axis_index("s") * NUM_CORES + lax.axis_index("c")
        pltpu.sync_copy(w_hbm, w_v)

        def do_unit(i, carry):
            u = wid * UNITS_PER_W + i
            s0 = (u // (DIM // DQ)) * SB
            d0 = (u % (DIM // DQ)) * DQ

            def build_row(si, c2):
                vec = jnp.maximum(w_v[s0 + si, pl.ds(d0, DQ)], 0.0)
                for di in range(DQ):
                    splat = jnp.full((LANES,), vec[di], dtype=jnp.float32)
                    for c in range(BW // LANES):
                        src[si, di, pl.ds(c * LANES, LANES)] = splat
                return c2

            lax.fori_loop(0, SB, build_row, 0)

            def ring(j, c2):
                pltpu.async_copy(
                    src,
                    out_hbm.at[pl.ds(s0, SB), pl.ds(d0, DQ),
                               pl.ds(j * BW, BW)],
                    sem)

                @pl.when(j >= FIRE)
                def _():
                    pltpu.make_async_copy(
                        src,
                        out_hbm.at[pl.ds(s0, SB), pl.ds(d0, DQ),
                                   pl.ds(0, BW)],
                        sem).wait()

                return c2

            lax.fori_loop(0, NCHUNKS, ring, 0)

            def drain(j, c2):
                pltpu.make_async_copy(
                    src,
                    out_hbm.at[pl.ds(s0, SB), pl.ds(d0, DQ), pl.ds(0, BW)],
                    sem).wait()
                return c2

            lax.fori_loop(0, FIRE, drain, 0)
            return carry

        lax.fori_loop(0, UNITS_PER_W, do_unit, 0)

    return k(w)


def kernel(x, embed_weight):
    seq = x.shape[1]
    out = _sc_pos_broadcast(embed_weight[:seq])
    return jnp.transpose(out, (2, 0, 1))
